# baseline (speedup 1.0000x reference)
import jax
import jax.numpy as jnp
from jax import lax
from jax.experimental import pallas as pl
from jax.experimental.pallas import tpu as pltpu
from jax.experimental.pallas import tpu_sc as plsc

N_VOCAB = 49408
D_EMBED = 768
N_TOKENS = 77
BATCH = 256

NC = 2
NS = 16
L = 16
NW = NC * NS
B_FLAT = BATCH * N_TOKENS
BPW = BATCH // NW
TPAD = 80
LANES_PER_ROW = D_EMBED // L

_MESH = dict(core_axis_name="c", subcore_axis_name="s", num_cores=NC,
             num_subcores=NS)


def _fast_body(table_hbm, tok_hbm, out_hbm,
               idx_v, stag0, stag1, gsem0, gsem1, osem0, osem1):
    wid = lax.axis_index("s") * NC + lax.axis_index("c")
    b0 = wid * BPW

    pltpu.sync_copy(tok_hbm.at[pl.ds(wid * (BPW * TPAD), BPW * TPAD)], idx_v)

    stags = (stag0, stag1)
    gsems = (gsem0, gsem1)
    osems = (osem0, osem1)

    def gather(bb, slot):
        off = pl.multiple_of(bb * TPAD, 8)
        pltpu.async_copy(table_hbm.at[idx_v.at[pl.ds(off, TPAD)]],
                         stags[slot], gsems[slot])

    gather(0, 0)

    def pair_body(pair, _):
        bb = 2 * pair

        @pl.when(pair >= 1)
        def _():
            pltpu.make_async_copy(stags[1], out_hbm.at[b0], osems[1]).wait()

        gather(bb + 1, 1)
        pltpu.make_async_copy(out_hbm.at[b0], stags[0], gsems[0]).wait()
        pltpu.async_copy(stags[0], out_hbm.at[b0 + bb], osems[0])

        @pl.when(pair < BPW // 2 - 1)
        def _():
            pltpu.make_async_copy(stags[0], out_hbm.at[b0], osems[0]).wait()
            gather(bb + 2, 0)

        pltpu.make_async_copy(out_hbm.at[b0], stags[1], gsems[1]).wait()
        pltpu.async_copy(stags[1], out_hbm.at[b0 + bb + 1], osems[1])
        return 0

    lax.fori_loop(0, BPW // 2, pair_body, 0)
    for s in range(2):
        pltpu.make_async_copy(stags[s], out_hbm.at[b0], osems[s]).wait()


def _fast(tok_pad, token_table):
    run = pl.kernel(
        _fast_body,
        out_type=jax.ShapeDtypeStruct((BATCH, TPAD, D_EMBED),
                                      jnp.float32),
        mesh=plsc.VectorSubcoreMesh(**_MESH),
        scratch_types=[
            pltpu.VMEM((BPW * TPAD,), jnp.int32),
            pltpu.VMEM((TPAD, D_EMBED), jnp.float32),
            pltpu.VMEM((TPAD, D_EMBED), jnp.float32),
            pltpu.SemaphoreType.DMA,
            pltpu.SemaphoreType.DMA,
            pltpu.SemaphoreType.DMA,
            pltpu.SemaphoreType.DMA,
        ],
    )
    return run(token_table, tok_pad)[:, :N_TOKENS, :]


ROWS_PER_W = B_FLAT // NW
CHUNK = 8
N_CHUNKS = ROWS_PER_W // CHUNK
NBUF = 7
N_GROUPS = N_CHUNKS // NBUF


def _slow_body(table_hbm, tok_hbm, pos_hbm, out_hbm, idx_v, pos_v, *rest):
    bufs = rest[:NBUF]
    gsems = rest[NBUF:2 * NBUF]
    osems = rest[2 * NBUF:3 * NBUF]

    wid = lax.axis_index("s") * NC + lax.axis_index("c")

    pltpu.sync_copy(tok_hbm.at[pl.ds(wid * ROWS_PER_W, ROWS_PER_W)], idx_v)
    pltpu.sync_copy(pos_hbm, pos_v)

    def gather(k, slot):
        off = pl.multiple_of(k * CHUNK, CHUNK)
        pltpu.async_copy(table_hbm.at[idx_v.at[pl.ds(off, CHUNK)]],
                         bufs[slot], gsems[slot])

    def add_pos(t0, slot):
        buf = bufs[slot]

        def row(j, t):
            base = pl.multiple_of(t * D_EMBED, L)
            for v in range(LANES_PER_ROW):
                vec = pos_v[pl.ds(base + v * L, L)]
                plsc.addupdate(buf.at[j, pl.ds(v * L, L)], vec)
            t = t + 1
            return jnp.where(t == N_TOKENS, 0, t)

        lax.fori_loop(0, CHUNK, row, t0)

    for s in range(NBUF):
        gather(s, s)

    def group(g, tg):
        for s in range(NBUF):
            k = g * NBUF + s
            nxt = k + NBUF - 1
            pslot = (s - 1) % NBUF

            @pl.when(jnp.logical_and(k >= 1, nxt < N_CHUNKS))
            def _():
                pltpu.make_async_copy(bufs[pslot], out_hbm.at[wid, 0],
                                      osems[pslot]).wait()
                gather(nxt, pslot)

            pltpu.make_async_copy(out_hbm.at[wid, 0], bufs[s],
                                  gsems[s]).wait()
            t0 = tg + (s * CHUNK) % N_TOKENS
            t0 = jnp.where(t0 >= N_TOKENS, t0 - N_TOKENS, t0)
            add_pos(t0, s)
            pltpu.async_copy(bufs[s], out_hbm.at[wid, k], osems[s])
        tg = tg + (NBUF * CHUNK) % N_TOKENS
        return jnp.where(tg >= N_TOKENS, tg - N_TOKENS, tg)

    lax.fori_loop(0, N_GROUPS, group, jnp.int32(0))

    for s in range(NBUF):
        pltpu.make_async_copy(bufs[s], out_hbm.at[wid, 0], osems[s]).wait()


def _slow(tok_flat, token_table, pos_flat):
    scratch = [
        pltpu.VMEM((ROWS_PER_W,), jnp.int32),
        pltpu.VMEM((N_TOKENS * D_EMBED,), jnp.float32),
    ]
    scratch += [pltpu.VMEM((CHUNK, D_EMBED), jnp.float32)
                for _ in range(NBUF)]
    scratch += [pltpu.SemaphoreType.DMA for _ in range(2 * NBUF)]
    run = pl.kernel(
        _slow_body,
        out_type=jax.ShapeDtypeStruct((NW, N_CHUNKS, CHUNK, D_EMBED),
                                      jnp.float32),
        mesh=plsc.VectorSubcoreMesh(**_MESH),
        scratch_types=scratch,
    )
    out = run(token_table, tok_flat, pos_flat)
    return out.reshape(BATCH, N_TOKENS, D_EMBED)


def kernel(tokens, token_table, pos_emb):
    tok = tokens.astype(jnp.int32)
    tok_pad = jnp.pad(tok, ((0, 0), (0, TPAD - N_TOKENS))).reshape(-1)
    tok_flat = tok.reshape(-1)
    pos_flat = pos_emb.reshape(-1)

    return lax.cond(
        jnp.any(pos_emb != 0.0),
        lambda: _slow(tok_flat, token_table, pos_flat),
        lambda: _fast(tok_pad, token_table),
    )


# speedup vs baseline: 1.0012x; 1.0012x over previous
import jax
import jax.numpy as jnp
from jax import lax
from jax.experimental import pallas as pl
from jax.experimental.pallas import tpu as pltpu
from jax.experimental.pallas import tpu_sc as plsc

N_VOCAB = 49408
D_EMBED = 768
N_TOKENS = 77
BATCH = 256

NC = 2
NS = 16
L = 16
NW = NC * NS
B_FLAT = BATCH * N_TOKENS
BPW = BATCH // NW
TPAD = 80
LANES_PER_ROW = D_EMBED // L

_MESH = dict(core_axis_name="c", subcore_axis_name="s", num_cores=NC,
             num_subcores=NS)


FCHUNK = 40
FNBUF = 4
F_CHUNKS = BPW * TPAD // FCHUNK
F_GROUPS = F_CHUNKS // FNBUF


def _fast_body(table_hbm, tok_hbm, out_hbm, idx_v, *rest):
    stags = rest[:FNBUF]
    gsems = rest[FNBUF:2 * FNBUF]
    osems = rest[2 * FNBUF:3 * FNBUF]

    wid = lax.axis_index("s") * NC + lax.axis_index("c")
    b0 = wid * BPW

    pltpu.sync_copy(tok_hbm.at[pl.ds(wid * (BPW * TPAD), BPW * TPAD)], idx_v)

    def gather(k, slot):
        off = pl.multiple_of(k * FCHUNK, 8)
        pltpu.async_copy(table_hbm.at[idx_v.at[pl.ds(off, FCHUNK)]],
                         stags[slot], gsems[slot])

    def store(k_bb, h, slot):
        pltpu.async_copy(
            stags[slot], out_hbm.at[b0 + k_bb, pl.ds(h * FCHUNK, FCHUNK)],
            osems[slot])

    for s in range(FNBUF):
        gather(s, s)

    def group(g, _):
        for s in range(FNBUF):
            k = g * FNBUF + s
            nxt = k + FNBUF - 1
            pslot = (s - 1) % FNBUF

            @pl.when(jnp.logical_and(k >= 1, nxt < F_CHUNKS))
            def _():
                pltpu.make_async_copy(
                    stags[pslot], out_hbm.at[b0, pl.ds(0, FCHUNK)],
                    osems[pslot]).wait()
                gather(nxt, pslot)

            pltpu.make_async_copy(out_hbm.at[b0, pl.ds(0, FCHUNK)],
                                  stags[s], gsems[s]).wait()
            store(2 * g + s // 2, s % 2, s)
        return 0

    lax.fori_loop(0, F_GROUPS, group, 0)
    for s in range(FNBUF):
        pltpu.make_async_copy(stags[s], out_hbm.at[b0, pl.ds(0, FCHUNK)],
                              osems[s]).wait()


def _fast(tok_pad, token_table):
    scratch = [pltpu.VMEM((BPW * TPAD,), jnp.int32)]
    scratch += [pltpu.VMEM((FCHUNK, D_EMBED), jnp.float32)
                for _ in range(FNBUF)]
    scratch += [pltpu.SemaphoreType.DMA for _ in range(2 * FNBUF)]
    run = pl.kernel(
        _fast_body,
        out_type=jax.ShapeDtypeStruct((BATCH, TPAD, D_EMBED),
                                      jnp.float32),
        mesh=plsc.VectorSubcoreMesh(**_MESH),
        scratch_types=scratch,
    )
    return run(token_table, tok_pad)[:, :N_TOKENS, :]


ROWS_PER_W = B_FLAT // NW
CHUNK = 8
N_CHUNKS = ROWS_PER_W // CHUNK
NBUF = 7
N_GROUPS = N_CHUNKS // NBUF


def _slow_body(table_hbm, tok_hbm, pos_hbm, out_hbm, idx_v, pos_v, *rest):
    bufs = rest[:NBUF]
    gsems = rest[NBUF:2 * NBUF]
    osems = rest[2 * NBUF:3 * NBUF]

    wid = lax.axis_index("s") * NC + lax.axis_index("c")

    pltpu.sync_copy(tok_hbm.at[pl.ds(wid * ROWS_PER_W, ROWS_PER_W)], idx_v)
    pltpu.sync_copy(pos_hbm, pos_v)

    def gather(k, slot):
        off = pl.multiple_of(k * CHUNK, CHUNK)
        pltpu.async_copy(table_hbm.at[idx_v.at[pl.ds(off, CHUNK)]],
                         bufs[slot], gsems[slot])

    def add_pos(t0, slot):
        buf = bufs[slot]

        def row(j, t):
            base = pl.multiple_of(t * D_EMBED, L)
            for v in range(LANES_PER_ROW):
                vec = pos_v[pl.ds(base + v * L, L)]
                plsc.addupdate(buf.at[j, pl.ds(v * L, L)], vec)
            t = t + 1
            return jnp.where(t == N_TOKENS, 0, t)

        lax.fori_loop(0, CHUNK, row, t0)

    for s in range(NBUF):
        gather(s, s)

    def group(g, tg):
        for s in range(NBUF):
            k = g * NBUF + s
            nxt = k + NBUF - 1
            pslot = (s - 1) % NBUF

            @pl.when(jnp.logical_and(k >= 1, nxt < N_CHUNKS))
            def _():
                pltpu.make_async_copy(bufs[pslot], out_hbm.at[wid, 0],
                                      osems[pslot]).wait()
                gather(nxt, pslot)

            pltpu.make_async_copy(out_hbm.at[wid, 0], bufs[s],
                                  gsems[s]).wait()
            t0 = tg + (s * CHUNK) % N_TOKENS
            t0 = jnp.where(t0 >= N_TOKENS, t0 - N_TOKENS, t0)
            add_pos(t0, s)
            pltpu.async_copy(bufs[s], out_hbm.at[wid, k], osems[s])
        tg = tg + (NBUF * CHUNK) % N_TOKENS
        return jnp.where(tg >= N_TOKENS, tg - N_TOKENS, tg)

    lax.fori_loop(0, N_GROUPS, group, jnp.int32(0))

    for s in range(NBUF):
        pltpu.make_async_copy(bufs[s], out_hbm.at[wid, 0], osems[s]).wait()


def _slow(tok_flat, token_table, pos_flat):
    scratch = [
        pltpu.VMEM((ROWS_PER_W,), jnp.int32),
        pltpu.VMEM((N_TOKENS * D_EMBED,), jnp.float32),
    ]
    scratch += [pltpu.VMEM((CHUNK, D_EMBED), jnp.float32)
                for _ in range(NBUF)]
    scratch += [pltpu.SemaphoreType.DMA for _ in range(2 * NBUF)]
    run = pl.kernel(
        _slow_body,
        out_type=jax.ShapeDtypeStruct((NW, N_CHUNKS, CHUNK, D_EMBED),
                                      jnp.float32),
        mesh=plsc.VectorSubcoreMesh(**_MESH),
        scratch_types=scratch,
    )
    out = run(token_table, tok_flat, pos_flat)
    return out.reshape(BATCH, N_TOKENS, D_EMBED)


def kernel(tokens, token_table, pos_emb):
    tok = tokens.astype(jnp.int32)
    tok_pad = jnp.pad(tok, ((0, 0), (0, TPAD - N_TOKENS))).reshape(-1)
    tok_flat = tok.reshape(-1)
    pos_flat = pos_emb.reshape(-1)

    return lax.cond(
        jnp.any(pos_emb != 0.0),
        lambda: _slow(tok_flat, token_table, pos_flat),
        lambda: _fast(tok_pad, token_table),
    )


# speedup vs baseline: 2.5821x; 2.5790x over previous
import jax
import jax.numpy as jnp
from jax import lax
from jax.experimental import pallas as pl
from jax.experimental.pallas import tpu as pltpu
from jax.experimental.pallas import tpu_sc as plsc

N_VOCAB = 49408
D_EMBED = 768
N_TOKENS = 77
BATCH = 256

NC = 2
NS = 16
L = 16
NW = NC * NS
B_FLAT = BATCH * N_TOKENS
BPW = BATCH // NW
TPAD = 80
LANES_PER_ROW = D_EMBED // L

_MESH = dict(core_axis_name="c", subcore_axis_name="s", num_cores=NC,
             num_subcores=NS)


FG = 7
FNBUF = 3
F_CHUNKS = N_TOKENS // FG
F_ROWS = FG * BPW


def _fast_body(table_hbm, tok_hbm, out_hbm, idx_v, *rest):
    stags = rest[:FNBUF]
    gsems = rest[FNBUF:2 * FNBUF]
    osems = rest[2 * FNBUF:3 * FNBUF]

    wid = lax.axis_index("s") * NC + lax.axis_index("c")
    c0 = wid * BPW

    pltpu.sync_copy(
        tok_hbm.at[pl.ds(wid * (BPW * N_TOKENS), BPW * N_TOKENS)], idx_v)

    def gather(k, slot):
        off = pl.multiple_of(k * F_ROWS, 8)
        pltpu.async_copy(table_hbm.at[idx_v.at[pl.ds(off, F_ROWS)]],
                         stags[slot], gsems[slot])

    def store(k, slot):
        for i in range(FG):
            pltpu.async_copy(
                stags[slot].at[pl.ds(i * BPW, BPW)],
                out_hbm.at[k * FG + i, pl.ds(c0, BPW)],
                osems[slot])

    def wait_store(slot):
        pltpu.make_async_copy(table_hbm.at[pl.ds(0, F_ROWS)], stags[slot],
                              osems[slot]).wait()

    def wait_gather(slot):
        pltpu.make_async_copy(table_hbm.at[pl.ds(0, F_ROWS)], stags[slot],
                              gsems[slot]).wait()

    for s in range(FNBUF):
        gather(s, s)

    def group(g, _):
        for s in range(FNBUF):
            k = g * FNBUF + s
            pslot = (s - 1) % FNBUF

            if s == 0:
                @pl.when(g >= 1)
                def _():
                    wait_store(pslot)
                    gather(k + FNBUF - 1, pslot)
            else:
                wait_store(pslot)
                gather(k + FNBUF - 1, pslot)

            wait_gather(s)
            store(k, s)
        return 0

    lax.fori_loop(0, F_CHUNKS // FNBUF, group, 0)
    for k in range(FNBUF * (F_CHUNKS // FNBUF), F_CHUNKS):
        s = k % FNBUF
        wait_gather(s)
        store(k, s)
    for s in range(FNBUF):
        wait_store(s)


def _fast(tok_w, token_table):
    scratch = [pltpu.VMEM((BPW * N_TOKENS,), jnp.int32)]
    scratch += [pltpu.VMEM((F_ROWS, D_EMBED), jnp.float32)
                for _ in range(FNBUF)]
    scratch += [pltpu.SemaphoreType.DMA for _ in range(2 * FNBUF)]
    run = pl.kernel(
        _fast_body,
        out_type=jax.ShapeDtypeStruct((N_TOKENS, BATCH, D_EMBED),
                                      jnp.float32),
        mesh=plsc.VectorSubcoreMesh(**_MESH),
        scratch_types=scratch,
    )
    return jnp.transpose(run(token_table, tok_w), (1, 0, 2))


ROWS_PER_W = B_FLAT // NW
CHUNK = 8
N_CHUNKS = ROWS_PER_W // CHUNK
NBUF = 7
N_GROUPS = N_CHUNKS // NBUF


def _slow_body(table_hbm, tok_hbm, pos_hbm, out_hbm, idx_v, pos_v, *rest):
    bufs = rest[:NBUF]
    gsems = rest[NBUF:2 * NBUF]
    osems = rest[2 * NBUF:3 * NBUF]

    wid = lax.axis_index("s") * NC + lax.axis_index("c")

    pltpu.sync_copy(tok_hbm.at[pl.ds(wid * ROWS_PER_W, ROWS_PER_W)], idx_v)
    pltpu.sync_copy(pos_hbm, pos_v)

    def gather(k, slot):
        off = pl.multiple_of(k * CHUNK, CHUNK)
        pltpu.async_copy(table_hbm.at[idx_v.at[pl.ds(off, CHUNK)]],
                         bufs[slot], gsems[slot])

    def add_pos(t0, slot):
        buf = bufs[slot]

        def row(j, t):
            base = pl.multiple_of(t * D_EMBED, L)
            for v in range(LANES_PER_ROW):
                vec = pos_v[pl.ds(base + v * L, L)]
                plsc.addupdate(buf.at[j, pl.ds(v * L, L)], vec)
            t = t + 1
            return jnp.where(t == N_TOKENS, 0, t)

        lax.fori_loop(0, CHUNK, row, t0)

    for s in range(NBUF):
        gather(s, s)

    def group(g, tg):
        for s in range(NBUF):
            k = g * NBUF + s
            nxt = k + NBUF - 1
            pslot = (s - 1) % NBUF

            @pl.when(jnp.logical_and(k >= 1, nxt < N_CHUNKS))
            def _():
                pltpu.make_async_copy(bufs[pslot], out_hbm.at[wid, 0],
                                      osems[pslot]).wait()
                gather(nxt, pslot)

            pltpu.make_async_copy(out_hbm.at[wid, 0], bufs[s],
                                  gsems[s]).wait()
            t0 = tg + (s * CHUNK) % N_TOKENS
            t0 = jnp.where(t0 >= N_TOKENS, t0 - N_TOKENS, t0)
            add_pos(t0, s)
            pltpu.async_copy(bufs[s], out_hbm.at[wid, k], osems[s])
        tg = tg + (NBUF * CHUNK) % N_TOKENS
        return jnp.where(tg >= N_TOKENS, tg - N_TOKENS, tg)

    lax.fori_loop(0, N_GROUPS, group, jnp.int32(0))

    for s in range(NBUF):
        pltpu.make_async_copy(bufs[s], out_hbm.at[wid, 0], osems[s]).wait()


def _slow(tok_flat, token_table, pos_flat):
    scratch = [
        pltpu.VMEM((ROWS_PER_W,), jnp.int32),
        pltpu.VMEM((N_TOKENS * D_EMBED,), jnp.float32),
    ]
    scratch += [pltpu.VMEM((CHUNK, D_EMBED), jnp.float32)
                for _ in range(NBUF)]
    scratch += [pltpu.SemaphoreType.DMA for _ in range(2 * NBUF)]
    run = pl.kernel(
        _slow_body,
        out_type=jax.ShapeDtypeStruct((NW, N_CHUNKS, CHUNK, D_EMBED),
                                      jnp.float32),
        mesh=plsc.VectorSubcoreMesh(**_MESH),
        scratch_types=scratch,
    )
    out = run(token_table, tok_flat, pos_flat)
    return out.reshape(BATCH, N_TOKENS, D_EMBED)


def kernel(tokens, token_table, pos_emb):
    tok = tokens.astype(jnp.int32)
    tok_w = tok.reshape(NW, BPW, N_TOKENS).transpose(0, 2, 1).reshape(-1)
    tok_flat = tok.reshape(-1)
    pos_flat = pos_emb.reshape(-1)

    return lax.cond(
        jnp.any(pos_emb != 0.0),
        lambda: _slow(tok_flat, token_table, pos_flat),
        lambda: _fast(tok_w, token_table),
    )


# speedup vs baseline: 2.6146x; 1.0126x over previous
import jax
import jax.numpy as jnp
from jax import lax
from jax.experimental import pallas as pl
from jax.experimental.pallas import tpu as pltpu
from jax.experimental.pallas import tpu_sc as plsc

N_VOCAB = 49408
D_EMBED = 768
N_TOKENS = 77
BATCH = 256

NC = 2
NS = 16
L = 16
NW = NC * NS
B_FLAT = BATCH * N_TOKENS
BPW = BATCH // NW
TPAD = 80
LANES_PER_ROW = D_EMBED // L

_MESH = dict(core_axis_name="c", subcore_axis_name="s", num_cores=NC,
             num_subcores=NS)


FG = 7
FNBUF = 3
F_CHUNKS = N_TOKENS // FG
F_ROWS = FG * BPW


def _fast_body(table_hbm, tok_hbm, out_hbm, idx_v, *rest):
    stags = rest[:FNBUF]
    gsems = rest[FNBUF:2 * FNBUF]
    osems = rest[2 * FNBUF:3 * FNBUF]

    wid = lax.axis_index("s") * NC + lax.axis_index("c")
    c0 = wid * BPW

    pltpu.sync_copy(
        tok_hbm.at[pl.ds(wid * (BPW * N_TOKENS), BPW * N_TOKENS)], idx_v)

    def gather(k, slot):
        off = pl.multiple_of(k * F_ROWS, 8)
        pltpu.async_copy(table_hbm.at[idx_v.at[pl.ds(off, F_ROWS)]],
                         stags[slot], gsems[slot])

    def store(k, slot):
        pltpu.async_copy(
            stags[slot].reshape(FG, BPW, D_EMBED),
            out_hbm.at[pl.ds(k * FG, FG), pl.ds(c0, BPW)],
            osems[slot])

    def wait_store(slot):
        pltpu.make_async_copy(table_hbm.at[pl.ds(0, F_ROWS)], stags[slot],
                              osems[slot]).wait()

    def wait_gather(slot):
        pltpu.make_async_copy(table_hbm.at[pl.ds(0, F_ROWS)], stags[slot],
                              gsems[slot]).wait()

    for s in range(FNBUF):
        gather(s, s)

    def group(g, _):
        for s in range(FNBUF):
            k = g * FNBUF + s
            pslot = (s - 1) % FNBUF

            if s == 0:
                @pl.when(g >= 1)
                def _():
                    wait_store(pslot)
                    gather(k + FNBUF - 1, pslot)
            else:
                wait_store(pslot)
                gather(k + FNBUF - 1, pslot)

            wait_gather(s)
            store(k, s)
        return 0

    lax.fori_loop(0, F_CHUNKS // FNBUF, group, 0)
    for k in range(FNBUF * (F_CHUNKS // FNBUF), F_CHUNKS):
        s = k % FNBUF
        wait_gather(s)
        store(k, s)
    for s in range(FNBUF):
        wait_store(s)


def _fast(tok_w, token_table):
    scratch = [pltpu.VMEM((BPW * N_TOKENS,), jnp.int32)]
    scratch += [pltpu.VMEM((F_ROWS, D_EMBED), jnp.float32)
                for _ in range(FNBUF)]
    scratch += [pltpu.SemaphoreType.DMA for _ in range(2 * FNBUF)]
    run = pl.kernel(
        _fast_body,
        out_type=jax.ShapeDtypeStruct((N_TOKENS, BATCH, D_EMBED),
                                      jnp.float32),
        mesh=plsc.VectorSubcoreMesh(**_MESH),
        scratch_types=scratch,
    )
    return jnp.transpose(run(token_table, tok_w), (1, 0, 2))


ROWS_PER_W = B_FLAT // NW
CHUNK = 8
N_CHUNKS = ROWS_PER_W // CHUNK
NBUF = 7
N_GROUPS = N_CHUNKS // NBUF


def _slow_body(table_hbm, tok_hbm, pos_hbm, out_hbm, idx_v, pos_v, *rest):
    bufs = rest[:NBUF]
    gsems = rest[NBUF:2 * NBUF]
    osems = rest[2 * NBUF:3 * NBUF]

    wid = lax.axis_index("s") * NC + lax.axis_index("c")

    pltpu.sync_copy(tok_hbm.at[pl.ds(wid * ROWS_PER_W, ROWS_PER_W)], idx_v)
    pltpu.sync_copy(pos_hbm, pos_v)

    def gather(k, slot):
        off = pl.multiple_of(k * CHUNK, CHUNK)
        pltpu.async_copy(table_hbm.at[idx_v.at[pl.ds(off, CHUNK)]],
                         bufs[slot], gsems[slot])

    def add_pos(t0, slot):
        buf = bufs[slot]

        def row(j, t):
            base = pl.multiple_of(t * D_EMBED, L)
            for v in range(LANES_PER_ROW):
                vec = pos_v[pl.ds(base + v * L, L)]
                plsc.addupdate(buf.at[j, pl.ds(v * L, L)], vec)
            t = t + 1
            return jnp.where(t == N_TOKENS, 0, t)

        lax.fori_loop(0, CHUNK, row, t0)

    for s in range(NBUF):
        gather(s, s)

    def group(g, tg):
        for s in range(NBUF):
            k = g * NBUF + s
            nxt = k + NBUF - 1
            pslot = (s - 1) % NBUF

            @pl.when(jnp.logical_and(k >= 1, nxt < N_CHUNKS))
            def _():
                pltpu.make_async_copy(bufs[pslot], out_hbm.at[wid, 0],
                                      osems[pslot]).wait()
                gather(nxt, pslot)

            pltpu.make_async_copy(out_hbm.at[wid, 0], bufs[s],
                                  gsems[s]).wait()
            t0 = tg + (s * CHUNK) % N_TOKENS
            t0 = jnp.where(t0 >= N_TOKENS, t0 - N_TOKENS, t0)
            add_pos(t0, s)
            pltpu.async_copy(bufs[s], out_hbm.at[wid, k], osems[s])
        tg = tg + (NBUF * CHUNK) % N_TOKENS
        return jnp.where(tg >= N_TOKENS, tg - N_TOKENS, tg)

    lax.fori_loop(0, N_GROUPS, group, jnp.int32(0))

    for s in range(NBUF):
        pltpu.make_async_copy(bufs[s], out_hbm.at[wid, 0], osems[s]).wait()


def _slow(tok_flat, token_table, pos_flat):
    scratch = [
        pltpu.VMEM((ROWS_PER_W,), jnp.int32),
        pltpu.VMEM((N_TOKENS * D_EMBED,), jnp.float32),
    ]
    scratch += [pltpu.VMEM((CHUNK, D_EMBED), jnp.float32)
                for _ in range(NBUF)]
    scratch += [pltpu.SemaphoreType.DMA for _ in range(2 * NBUF)]
    run = pl.kernel(
        _slow_body,
        out_type=jax.ShapeDtypeStruct((NW, N_CHUNKS, CHUNK, D_EMBED),
                                      jnp.float32),
        mesh=plsc.VectorSubcoreMesh(**_MESH),
        scratch_types=scratch,
    )
    out = run(token_table, tok_flat, pos_flat)
    return out.reshape(BATCH, N_TOKENS, D_EMBED)


def kernel(tokens, token_table, pos_emb):
    tok = tokens.astype(jnp.int32)
    tok_w = tok.reshape(NW, BPW, N_TOKENS).transpose(0, 2, 1).reshape(-1)
    tok_flat = tok.reshape(-1)
    pos_flat = pos_emb.reshape(-1)

    return lax.cond(
        jnp.any(pos_emb != 0.0),
        lambda: _slow(tok_flat, token_table, pos_flat),
        lambda: _fast(tok_w, token_table),
    )
